# Initial kernel scaffold; baseline (speedup 1.0000x reference)
import functools

import jax
import jax.numpy as jnp
from jax import lax
from jax.experimental import pallas as pl
from jax.experimental.pallas import tpu as pltpu
from jax.experimental.pallas import tpu_sc as plsc

NN = 100000
EE = 3200000
NC = 2
NS = 16
LL = 16

EB = 4000
G = 128


def _sc_mesh():
    return plsc.VectorSubcoreMesh(core_axis_name="c", subcore_axis_name="s")


def _hist_body(dst_hbm, out_hbm, hist, dbuf):
    c_id = lax.axis_index("c")
    s_id = lax.axis_index("s")
    wid = s_id * NC + c_id
    zero = jnp.zeros((LL,), jnp.float32)
    one = jnp.ones((LL,), jnp.float32)

    def _z(i, _):
        hist[pl.ds(i * LL, LL)] = zero
        return 0
    lax.fori_loop(0, NN // LL, _z, 0)

    per_tile = EE // (NC * NS)

    def _blk(b, _):
        base = wid * per_tile + b * EB
        pltpu.sync_copy(dst_hbm.at[pl.ds(base, EB)], dbuf)

        def _vec(j, _):
            idx = dbuf[pl.ds(j * LL, LL)]
            plsc.addupdate_scatter(hist, [idx], one)
            return 0
        lax.fori_loop(0, EB // LL, _vec, 0)
        return 0
    lax.fori_loop(0, per_tile // EB, _blk, 0)

    pltpu.sync_copy(hist, out_hbm.at[wid])


def _hist(dst):
    f = pl.kernel(
        _hist_body,
        out_type=jax.ShapeDtypeStruct((NC * NS, NN), jnp.float32),
        mesh=_sc_mesh(),
        scratch_types=[
            pltpu.VMEM((NN,), jnp.float32),
            pltpu.VMEM((EB,), jnp.int32),
        ],
    )
    return f(dst)


def _spmm_body(wp, C, nchunks, g_hbm, src_hbm, dst_hbm, out_hbm,
               acc, dbuf, sbuf, csrc, cdst, fsrc, fdst, rows, zbuf):
    c_id = lax.axis_index("c")
    s_id = lax.axis_index("s")
    rpt = C // NS
    per_tile = EE // NS
    izero = jnp.zeros((LL,), jnp.int32)
    lanes = lax.broadcasted_iota(jnp.int32, (LL,), 0)

    def _zz(i, _):
        r = i // (wp // LL)
        k = i % (wp // LL)
        zbuf.at[r][pl.ds(k * LL, LL)] = jnp.zeros((LL,), jnp.float32)
        return 0
    lax.fori_loop(0, (128 * wp) // LL, _zz, 0)

    def _flush(p):
        for k in range(G // LL):
            fsrc[pl.ds(k * LL, LL)] = csrc[pl.ds(k * LL, LL)]
            fdst[pl.ds(k * LL, LL)] = cdst[pl.ds(k * LL, LL)]
        pltpu.sync_copy(g_hbm.at[fsrc], rows)
        pltpu.sync_copy(rows, acc.at[fdst], add=True)
        return p

    def _chunk(i, _):
        chunk = i * NC + c_id
        lo = chunk * C

        nz = (rpt + 127) // 128
        for zi in range(nz):
            cnt = min(128, rpt - zi * 128)
            pltpu.sync_copy(zbuf.at[pl.ds(0, cnt)],
                            acc.at[pl.ds(s_id * rpt + zi * 128, cnt)])
        @pl.when(s_id == 0)
        def _():
            pltpu.sync_copy(zbuf.at[pl.ds(0, LL)], acc.at[pl.ds(C, LL)])
        plsc.subcore_barrier()

        def _blk(b, ptr):
            base = s_id * per_tile + b * EB
            pltpu.sync_copy(dst_hbm.at[pl.ds(base, EB)], dbuf)
            pltpu.sync_copy(src_hbm.at[pl.ds(base, EB)], sbuf)

            def _vec(j, p):
                d = dbuf[pl.ds(j * LL, LL)]
                s = sbuf[pl.ds(j * LL, LL)]
                m = (d >= lo) & (d < lo + C)
                plsc.store_compressed(csrc.at[pl.ds(p, LL)], s, mask=m)
                plsc.store_compressed(cdst.at[pl.ds(p, LL)], d - lo, mask=m)
                p = p + jnp.sum(m.astype(jnp.int32))

                def _do(q):
                    q = _flush(q)
                    csrc[pl.ds(0, LL)] = csrc[pl.ds(G, LL)]
                    cdst[pl.ds(0, LL)] = cdst[pl.ds(G, LL)]
                    return q - G
                return lax.cond(p >= G, _do, lambda q: q, p)
            return lax.fori_loop(0, EB // LL, _vec, ptr)
        ptr = lax.fori_loop(0, per_tile // EB, _blk, 0)

        def _pad(jj, _):
            keep = (lanes + jj * LL) < ptr
            sv = csrc[pl.ds(jj * LL, LL)]
            dv = cdst[pl.ds(jj * LL, LL)]
            csrc[pl.ds(jj * LL, LL)] = jnp.where(keep, sv, izero)
            cdst[pl.ds(jj * LL, LL)] = jnp.where(keep, dv, izero + C)
            return 0
        lax.fori_loop(0, G // LL, _pad, 0)
        _flush(0)

        plsc.subcore_barrier()
        pltpu.sync_copy(acc.at[pl.ds(s_id * rpt, rpt)],
                        out_hbm.at[pl.ds(lo + s_id * rpt, rpt)])
        plsc.subcore_barrier()
        return 0

    my_chunks = (nchunks + 1 - c_id) // NC
    lax.fori_loop(0, my_chunks, _chunk, 0)


def _spmm(g, src, dst, wp, C, nchunks):
    f = pl.kernel(
        functools.partial(_spmm_body, wp, C, nchunks),
        out_type=jax.ShapeDtypeStruct((nchunks * C, wp), jnp.float32),
        mesh=_sc_mesh(),
        scratch_types=[
            pltpu.VMEM_SHARED((C + LL, wp), jnp.float32),
            pltpu.VMEM((EB,), jnp.int32),
            pltpu.VMEM((EB,), jnp.int32),
            pltpu.VMEM((G + LL,), jnp.int32),
            pltpu.VMEM((G + LL,), jnp.int32),
            pltpu.VMEM((G,), jnp.int32),
            pltpu.VMEM((G,), jnp.int32),
            pltpu.VMEM((G, wp), jnp.float32),
            pltpu.VMEM((128, wp), jnp.float32),
        ],
    )
    return f(g, src, dst)


_R = 2000


def _reduce_body(h_ref, o_ref):
    deg = jnp.sum(h_ref[...], axis=0, keepdims=True) + 1.0
    o_ref[...] = lax.rsqrt(deg)


def _reduce_dinv(hists):
    out = pl.pallas_call(
        _reduce_body,
        out_shape=jax.ShapeDtypeStruct((1, NN), jnp.float32),
        grid=(NN // _R,),
        in_specs=[pl.BlockSpec((NC * NS, _R), lambda i: (0, i))],
        out_specs=pl.BlockSpec((1, _R), lambda i: (0, i)),
    )(hists)
    return jnp.reshape(out, (NN, 1))


def _l1_body(x_ref, d_ref, w_ref, o_ref):
    o_ref[...] = d_ref[...] * jnp.dot(
        x_ref[...], w_ref[...], preferred_element_type=jnp.float32)


def _layer1(xp, dinv, Wp, wp_in, wp_out):
    return pl.pallas_call(
        _l1_body,
        out_shape=jax.ShapeDtypeStruct((NN, wp_out), jnp.float32),
        grid=(NN // _R,),
        in_specs=[
            pl.BlockSpec((_R, wp_in), lambda i: (i, 0)),
            pl.BlockSpec((_R, 1), lambda i: (i, 0)),
            pl.BlockSpec((wp_in, wp_out), lambda i: (0, 0)),
        ],
        out_specs=pl.BlockSpec((_R, wp_out), lambda i: (i, 0)),
    )(xp, dinv, Wp)


def _mid_body(s_ref, g_ref, d_ref, b_ref, w_ref, o_ref):
    d = d_ref[...]
    pre = jnp.maximum(d * (s_ref[...] + g_ref[...]) + b_ref[...], 0.0)
    o_ref[...] = d * jnp.dot(pre, w_ref[...],
                             preferred_element_type=jnp.float32)


def _layer_mid(s, g, dinv, bp, Wp, wp_in, wp_out):
    return pl.pallas_call(
        _mid_body,
        out_shape=jax.ShapeDtypeStruct((NN, wp_out), jnp.float32),
        grid=(NN // _R,),
        in_specs=[
            pl.BlockSpec((_R, wp_in), lambda i: (i, 0)),
            pl.BlockSpec((_R, wp_in), lambda i: (i, 0)),
            pl.BlockSpec((_R, 1), lambda i: (i, 0)),
            pl.BlockSpec((1, wp_in), lambda i: (0, 0)),
            pl.BlockSpec((wp_in, wp_out), lambda i: (0, 0)),
        ],
        out_specs=pl.BlockSpec((_R, wp_out), lambda i: (i, 0)),
    )(s, g, dinv, bp, Wp)


def _final_body(w, s_ref, g_ref, d_ref, b_ref, z_ref, p_ref):
    z = jnp.maximum(d_ref[...] * (s_ref[...] + g_ref[...]) + b_ref[...], 0.0)
    wp = z.shape[1]
    msk = lax.broadcasted_iota(jnp.int32, (_R, wp), 1) < w
    zm = jnp.max(jnp.where(msk, z, -jnp.inf), axis=1, keepdims=True)
    ssum = jnp.sum(jnp.where(msk, jnp.exp(z - zm), 0.0), axis=1,
                   keepdims=True)
    z_ref[...] = z
    p_ref[...] = z - zm - jnp.log(ssum)


def _final(s, g, dinv, bp, w, wp_in):
    return pl.pallas_call(
        functools.partial(_final_body, w),
        out_shape=(jax.ShapeDtypeStruct((NN, wp_in), jnp.float32),
                   jax.ShapeDtypeStruct((NN, wp_in), jnp.float32)),
        grid=(NN // _R,),
        in_specs=[
            pl.BlockSpec((_R, wp_in), lambda i: (i, 0)),
            pl.BlockSpec((_R, wp_in), lambda i: (i, 0)),
            pl.BlockSpec((_R, 1), lambda i: (i, 0)),
            pl.BlockSpec((1, wp_in), lambda i: (0, 0)),
        ],
        out_specs=(pl.BlockSpec((_R, wp_in), lambda i: (i, 0)),
                   pl.BlockSpec((_R, wp_in), lambda i: (i, 0))),
    )(s, g, dinv, bp)


def _pad2(a, r, c):
    out = jnp.zeros((r, c), a.dtype)
    return out.at[:a.shape[0], :a.shape[1]].set(a)


_SPMM_GEOM = {176: (10000, 10), 96: (20000, 5), 48: (33344, 3),
              32: (50000, 2)}


def kernel(x, edge_index, train_flag, W1, b1, W3, b3, W4, b4, W2, b2):
    src = edge_index[0]
    dst = edge_index[1]

    hists = _hist(dst)
    dinv = _reduce_dinv(hists)

    c = 21
    dims = [21, 8 * c, 4 * c, 2 * c, c]
    wps = [32, 176, 96, 48, 32]
    Ws = [W1, W3, W4, W2]
    bs = [b1, b3, b4, b2]

    xp = _pad2(x, NN, wps[0])
    Wp = [_pad2(Ws[i], wps[i], wps[i + 1]) for i in range(4)]
    bp = [_pad2(bs[i][None, :], 1, wps[i + 1]) for i in range(4)]

    g = _layer1(xp, dinv, Wp[0], wps[0], wps[1])
    for li in (1, 2, 3):
        wp_in = wps[li]
        C, nch = _SPMM_GEOM[wp_in]
        s = _spmm(g, src, dst, wp_in, C, nch)[:NN]
        g = _layer_mid(s, g, dinv, bp[li - 1], Wp[li], wp_in, wps[li + 1])

    wp_in = wps[4]
    C, nch = _SPMM_GEOM[wp_in]
    s = _spmm(g, src, dst, wp_in, C, nch)[:NN]
    zfull, pfull = _final(s, g, dinv, bp[3], dims[4], wp_in)

    z = zfull[:, :dims[4]]
    p_z = pfull[:, :dims[4]]
    return (p_z, z, edge_index)


# speedup vs baseline: 7.9621x; 7.9621x over previous
import functools

import jax
import jax.numpy as jnp
from jax import lax
from jax.experimental import pallas as pl
from jax.experimental.pallas import tpu as pltpu
from jax.experimental.pallas import tpu_sc as plsc

NN = 100000
EE = 3200000
NC = 2
NS = 16
LL = 16

EB = 2000
HEB = 4000
G = 128


def _sc_mesh():
    return plsc.VectorSubcoreMesh(core_axis_name="c", subcore_axis_name="s")


def _hist_body(dst_hbm, out_hbm, hist, dbuf):
    c_id = lax.axis_index("c")
    s_id = lax.axis_index("s")
    wid = s_id * NC + c_id
    zero = jnp.zeros((LL,), jnp.float32)
    one = jnp.ones((LL,), jnp.float32)

    def _z(i, _):
        hist[pl.ds(i * LL, LL)] = zero
        return 0
    lax.fori_loop(0, NN // LL, _z, 0)

    per_tile = EE // (NC * NS)

    def _blk(b, _):
        base = wid * per_tile + b * HEB
        pltpu.sync_copy(dst_hbm.at[pl.ds(base, HEB)], dbuf)

        def _vec(j, _):
            idx = dbuf[pl.ds(j * LL, LL)]
            plsc.addupdate_scatter(hist, [idx], one)
            return 0
        lax.fori_loop(0, HEB // LL, _vec, 0)
        return 0
    lax.fori_loop(0, per_tile // HEB, _blk, 0)

    pltpu.sync_copy(hist, out_hbm.at[wid])


def _hist(dst):
    f = pl.kernel(
        _hist_body,
        out_type=jax.ShapeDtypeStruct((NC * NS, NN), jnp.float32),
        mesh=_sc_mesh(),
        scratch_types=[
            pltpu.VMEM((NN,), jnp.float32),
            pltpu.VMEM((HEB,), jnp.int32),
        ],
        compiler_params=pltpu.CompilerParams(needs_layout_passes=False, use_tc_tiling_on_sc=False),
    )
    return f(dst)


def _spmm_body(wp, C, nchunks, g_hbm, src_hbm, dst_hbm, out_hbm,
               acc, dbuf, sbuf, csrc, cdst, fsrc, fdst, rows, zbuf):
    c_id = lax.axis_index("c")
    s_id = lax.axis_index("s")
    rpt = C // NS
    per_tile = EE // NS
    izero = jnp.zeros((LL,), jnp.int32)
    lanes = lax.broadcasted_iota(jnp.int32, (LL,), 0)

    def _zz(i, _):
        r = i // (wp // LL)
        k = i % (wp // LL)
        zbuf.at[r][pl.ds(k * LL, LL)] = jnp.zeros((LL,), jnp.float32)
        return 0
    lax.fori_loop(0, (32 * wp) // LL, _zz, 0)

    def _flush(p):
        for k in range(G // LL):
            fsrc[pl.ds(k * LL, LL)] = csrc[pl.ds(k * LL, LL)]
            fdst[pl.ds(k * LL, LL)] = cdst[pl.ds(k * LL, LL)]
        pltpu.sync_copy(g_hbm.at[fsrc], rows)
        pltpu.sync_copy(rows, acc.at[fdst], add=True)
        return p

    def _chunk(i, _):
        chunk = i * NC + c_id
        lo = chunk * C

        nz = (rpt + 31) // 32
        for zi in range(nz):
            cnt = min(32, rpt - zi * 32)
            pltpu.sync_copy(zbuf.at[pl.ds(0, cnt)],
                            acc.at[pl.ds(s_id * rpt + zi * 32, cnt)])
        @pl.when(s_id == 0)
        def _():
            pltpu.sync_copy(zbuf.at[pl.ds(0, LL)], acc.at[pl.ds(C, LL)])
        plsc.subcore_barrier()

        def _blk(b, ptr):
            base = s_id * per_tile + b * EB
            pltpu.sync_copy(dst_hbm.at[pl.ds(base, EB)], dbuf)
            pltpu.sync_copy(src_hbm.at[pl.ds(base, EB)], sbuf)

            def _vec(j, p):
                d = dbuf[pl.ds(j * LL, LL)]
                s = sbuf[pl.ds(j * LL, LL)]
                m = (d >= lo) & (d < lo + C)
                plsc.store_compressed(csrc.at[pl.ds(p, LL)], s, mask=m)
                plsc.store_compressed(cdst.at[pl.ds(p, LL)], d - lo, mask=m)
                p = p + jnp.sum(m.astype(jnp.int32))

                def _do(q):
                    q = _flush(q)
                    csrc[pl.ds(0, LL)] = csrc[pl.ds(G, LL)]
                    cdst[pl.ds(0, LL)] = cdst[pl.ds(G, LL)]
                    return q - G
                return lax.cond(p >= G, _do, lambda q: q, p)
            return lax.fori_loop(0, EB // LL, _vec, ptr)
        ptr = lax.fori_loop(0, per_tile // EB, _blk, 0)

        def _pad(jj, _):
            keep = (lanes + jj * LL) < ptr
            sv = csrc[pl.ds(jj * LL, LL)]
            dv = cdst[pl.ds(jj * LL, LL)]
            csrc[pl.ds(jj * LL, LL)] = jnp.where(keep, sv, izero)
            cdst[pl.ds(jj * LL, LL)] = jnp.where(keep, dv, izero + C)
            return 0
        lax.fori_loop(0, G // LL, _pad, 0)
        _flush(0)

        plsc.subcore_barrier()
        pltpu.sync_copy(acc.at[pl.ds(s_id * rpt, rpt)],
                        out_hbm.at[pl.ds(lo + s_id * rpt, rpt)])
        plsc.subcore_barrier()
        return 0

    my_chunks = (nchunks + 1 - c_id) // NC
    lax.fori_loop(0, my_chunks, _chunk, 0)


def _spmm(g, src, dst, wp, C, nchunks):
    f = pl.kernel(
        functools.partial(_spmm_body, wp, C, nchunks),
        out_type=jax.ShapeDtypeStruct((nchunks * C, wp), jnp.float32),
        mesh=_sc_mesh(),
        scratch_types=[
            pltpu.VMEM_SHARED((C + LL, wp), jnp.float32),
            pltpu.VMEM((EB,), jnp.int32),
            pltpu.VMEM((EB,), jnp.int32),
            pltpu.VMEM((G + LL,), jnp.int32),
            pltpu.VMEM((G + LL,), jnp.int32),
            pltpu.VMEM((G,), jnp.int32),
            pltpu.VMEM((G,), jnp.int32),
            pltpu.VMEM((G, wp), jnp.float32),
            pltpu.VMEM((32, wp), jnp.float32),
        ],
        compiler_params=pltpu.CompilerParams(needs_layout_passes=False, use_tc_tiling_on_sc=False),
    )
    return f(g, src, dst)


_R = 2000


def _reduce_body(h_ref, o_ref):
    deg = jnp.sum(h_ref[...], axis=0, keepdims=True) + 1.0
    o_ref[...] = lax.rsqrt(deg)


def _reduce_dinv(hists):
    out = pl.pallas_call(
        _reduce_body,
        out_shape=jax.ShapeDtypeStruct((1, NN), jnp.float32),
    )(hists)
    return jnp.reshape(out, (NN, 1))


def _l1_body(x_ref, d_ref, w_ref, o_ref):
    o_ref[...] = d_ref[...] * jnp.dot(
        x_ref[...], w_ref[...], preferred_element_type=jnp.float32)


def _layer1(xp, dinv, Wp, wp_in, wp_out):
    return pl.pallas_call(
        _l1_body,
        out_shape=jax.ShapeDtypeStruct((NN, wp_out), jnp.float32),
        grid=(NN // _R,),
        in_specs=[
            pl.BlockSpec((_R, wp_in), lambda i: (i, 0)),
            pl.BlockSpec((_R, 1), lambda i: (i, 0)),
            pl.BlockSpec((wp_in, wp_out), lambda i: (0, 0)),
        ],
        out_specs=pl.BlockSpec((_R, wp_out), lambda i: (i, 0)),
    )(xp, dinv, Wp)


def _mid_body(s_ref, g_ref, d_ref, b_ref, w_ref, o_ref):
    d = d_ref[...]
    pre = jnp.maximum(d * (s_ref[...] + g_ref[...]) + b_ref[...], 0.0)
    o_ref[...] = d * jnp.dot(pre, w_ref[...],
                             preferred_element_type=jnp.float32)


def _layer_mid(s, g, dinv, bp, Wp, wp_in, wp_out):
    return pl.pallas_call(
        _mid_body,
        out_shape=jax.ShapeDtypeStruct((NN, wp_out), jnp.float32),
        grid=(NN // _R,),
        in_specs=[
            pl.BlockSpec((_R, wp_in), lambda i: (i, 0)),
            pl.BlockSpec((_R, wp_in), lambda i: (i, 0)),
            pl.BlockSpec((_R, 1), lambda i: (i, 0)),
            pl.BlockSpec((1, wp_in), lambda i: (0, 0)),
            pl.BlockSpec((wp_in, wp_out), lambda i: (0, 0)),
        ],
        out_specs=pl.BlockSpec((_R, wp_out), lambda i: (i, 0)),
    )(s, g, dinv, bp, Wp)


def _final_body(w, s_ref, g_ref, d_ref, b_ref, z_ref, p_ref):
    z = jnp.maximum(d_ref[...] * (s_ref[...] + g_ref[...]) + b_ref[...], 0.0)
    wp = z.shape[1]
    msk = lax.broadcasted_iota(jnp.int32, (_R, wp), 1) < w
    zm = jnp.max(jnp.where(msk, z, -jnp.inf), axis=1, keepdims=True)
    ssum = jnp.sum(jnp.where(msk, jnp.exp(z - zm), 0.0), axis=1,
                   keepdims=True)
    z_ref[...] = z
    p_ref[...] = z - zm - jnp.log(ssum)


def _final(s, g, dinv, bp, w, wp_in):
    return pl.pallas_call(
        functools.partial(_final_body, w),
        out_shape=(jax.ShapeDtypeStruct((NN, wp_in), jnp.float32),
                   jax.ShapeDtypeStruct((NN, wp_in), jnp.float32)),
        grid=(NN // _R,),
        in_specs=[
            pl.BlockSpec((_R, wp_in), lambda i: (i, 0)),
            pl.BlockSpec((_R, wp_in), lambda i: (i, 0)),
            pl.BlockSpec((_R, 1), lambda i: (i, 0)),
            pl.BlockSpec((1, wp_in), lambda i: (0, 0)),
        ],
        out_specs=(pl.BlockSpec((_R, wp_in), lambda i: (i, 0)),
                   pl.BlockSpec((_R, wp_in), lambda i: (i, 0))),
    )(s, g, dinv, bp)


def _pad2(a, r, c):
    out = jnp.zeros((r, c), a.dtype)
    return out.at[:a.shape[0], :a.shape[1]].set(a)


_SPMM_GEOM = {176: (8704, 12), 96: (18176, 6), 48: (35840, 3),
              32: (50048, 2)}


def kernel(x, edge_index, train_flag, W1, b1, W3, b3, W4, b4, W2, b2):
    src = edge_index[0]
    dst = edge_index[1]

    hists = _hist(dst)
    dinv = _reduce_dinv(hists)

    c = 21
    dims = [21, 8 * c, 4 * c, 2 * c, c]
    wps = [32, 176, 96, 48, 32]
    Ws = [W1, W3, W4, W2]
    bs = [b1, b3, b4, b2]

    xp = _pad2(x, NN, wps[0])
    Wp = [_pad2(Ws[i], wps[i], wps[i + 1]) for i in range(4)]
    bp = [_pad2(bs[i][None, :], 1, wps[i + 1]) for i in range(4)]

    g = _layer1(xp, dinv, Wp[0], wps[0], wps[1])
    for li in (1, 2, 3):
        wp_in = wps[li]
        C, nch = _SPMM_GEOM[wp_in]
        s = _spmm(g, src, dst, wp_in, C, nch)[:NN]
        g = _layer_mid(s, g, dinv, bp[li - 1], Wp[li], wp_in, wps[li + 1])

    wp_in = wps[4]
    C, nch = _SPMM_GEOM[wp_in]
    s = _spmm(g, src, dst, wp_in, C, nch)[:NN]
    zfull, pfull = _final(s, g, dinv, bp[3], dims[4], wp_in)

    z = zfull[:, :dims[4]]
    p_z = pfull[:, :dims[4]]
    return (p_z, z, edge_index)


# speedup vs baseline: 12.5943x; 1.5818x over previous
import functools

import jax
import jax.numpy as jnp
from jax import lax
from jax.experimental import pallas as pl
from jax.experimental.pallas import tpu as pltpu
from jax.experimental.pallas import tpu_sc as plsc

NN = 100000
EE = 3200000
NC = 2
NS = 16
LL = 16

EB = 1600
HEB = 4000
G = 128


def _sc_mesh():
    return plsc.VectorSubcoreMesh(core_axis_name="c", subcore_axis_name="s")


def _hist_body(dst_hbm, out_hbm, hist, dbuf):
    c_id = lax.axis_index("c")
    s_id = lax.axis_index("s")
    wid = s_id * NC + c_id
    zero = jnp.zeros((LL,), jnp.float32)
    one = jnp.ones((LL,), jnp.float32)

    def _z(i, _):
        hist[pl.ds(i * LL, LL)] = zero
        return 0
    lax.fori_loop(0, NN // LL, _z, 0)

    per_tile = EE // (NC * NS)

    def _blk(b, _):
        base = wid * per_tile + b * HEB
        pltpu.sync_copy(dst_hbm.at[pl.ds(base, HEB)], dbuf)

        def _vec(j, _):
            idx = dbuf[pl.ds(j * LL, LL)]
            plsc.addupdate_scatter(hist, [idx], one)
            return 0
        lax.fori_loop(0, HEB // LL, _vec, 0)
        return 0
    lax.fori_loop(0, per_tile // HEB, _blk, 0)

    pltpu.sync_copy(hist, out_hbm.at[wid])


def _hist(dst):
    f = pl.kernel(
        _hist_body,
        out_type=jax.ShapeDtypeStruct((NC * NS, NN), jnp.float32),
        mesh=_sc_mesh(),
        scratch_types=[
            pltpu.VMEM((NN,), jnp.float32),
            pltpu.VMEM((HEB,), jnp.int32),
        ],
        compiler_params=pltpu.CompilerParams(needs_layout_passes=False, use_tc_tiling_on_sc=False),
    )
    return f(dst)


def _spmm_body(wp, C, nchunks, g_hbm, src_hbm, dst_hbm, out_hbm,
               acc, dbuf, sbuf, csrc, cdst, fsrc, fdst, rows, zbuf,
               gsem, ssem):
    c_id = lax.axis_index("c")
    s_id = lax.axis_index("s")
    rpt = C // NS
    per_tile = EE // NS
    izero = jnp.zeros((LL,), jnp.int32)
    lanes = lax.broadcasted_iota(jnp.int32, (LL,), 0)

    def _zz(i, _):
        r = i // (wp // LL)
        k = i % (wp // LL)
        zbuf.at[r][pl.ds(k * LL, LL)] = jnp.zeros((LL,), jnp.float32)
        return 0
    lax.fori_loop(0, (16 * wp) // LL, _zz, 0)

    def _wait_gather():
        pltpu.make_async_copy(g_hbm.at[fsrc.at[0]], rows.at[0], gsem).wait()

    def _wait_scatter():
        pltpu.make_async_copy(rows.at[0], acc.at[fdst.at[0]], ssem).wait()

    def _start_batch(fc):
        p = fc & 1
        @pl.when(fc >= 2)
        def _():
            _wait_scatter()
        @pl.when(fc >= 1)
        def _():
            _wait_gather()
            q = 1 - p
            pltpu.make_async_copy(rows.at[q], acc.at[fdst.at[q]],
                                  ssem).start(add=True)
        for k in range(G // LL):
            fsrc.at[p][pl.ds(k * LL, LL)] = csrc[pl.ds(k * LL, LL)]
            fdst.at[p][pl.ds(k * LL, LL)] = cdst[pl.ds(k * LL, LL)]
        pltpu.make_async_copy(g_hbm.at[fsrc.at[p]], rows.at[p],
                              gsem).start()

    def _chunk(i, _):
        chunk = i * NC + c_id
        lo = chunk * C

        nz = (rpt + 15) // 16
        for zi in range(nz):
            cnt = min(16, rpt - zi * 16)
            pltpu.sync_copy(zbuf.at[pl.ds(0, cnt)],
                            acc.at[pl.ds(s_id * rpt + zi * 16, cnt)])
        @pl.when(s_id == 0)
        def _():
            pltpu.sync_copy(zbuf.at[pl.ds(0, LL)], acc.at[pl.ds(C, LL)])
        plsc.subcore_barrier()

        def _blk(b, carry):
            base = s_id * per_tile + b * EB
            pltpu.sync_copy(dst_hbm.at[pl.ds(base, EB)], dbuf)
            pltpu.sync_copy(src_hbm.at[pl.ds(base, EB)], sbuf)

            def _vec(j, carry):
                p, fc = carry
                d = dbuf[pl.ds(j * LL, LL)]
                s = sbuf[pl.ds(j * LL, LL)]
                m = (d >= lo) & (d < lo + C)
                plsc.store_compressed(csrc.at[pl.ds(p, LL)], s, mask=m)
                plsc.store_compressed(cdst.at[pl.ds(p, LL)], d - lo, mask=m)
                p = p + jnp.sum(m.astype(jnp.int32))

                def _do(q, fc):
                    _start_batch(fc)
                    csrc[pl.ds(0, LL)] = csrc[pl.ds(G, LL)]
                    cdst[pl.ds(0, LL)] = cdst[pl.ds(G, LL)]
                    return q - G, fc + 1
                return lax.cond(p >= G, _do, lambda q, fc: (q, fc), p, fc)
            return lax.fori_loop(0, EB // LL, _vec, carry)
        ptr, fc = lax.fori_loop(0, per_tile // EB, _blk, (0, 0))

        def _pad(jj, _):
            keep = (lanes + jj * LL) < ptr
            sv = csrc[pl.ds(jj * LL, LL)]
            dv = cdst[pl.ds(jj * LL, LL)]
            csrc[pl.ds(jj * LL, LL)] = jnp.where(keep, sv, izero)
            cdst[pl.ds(jj * LL, LL)] = jnp.where(keep, dv, izero + C)
            return 0
        lax.fori_loop(0, G // LL, _pad, 0)
        _start_batch(fc)
        @pl.when(fc >= 1)
        def _():
            _wait_scatter()
        _wait_gather()
        pf = fc & 1
        pltpu.make_async_copy(rows.at[pf], acc.at[fdst.at[pf]],
                              ssem).start(add=True)
        _wait_scatter()

        plsc.subcore_barrier()
        pltpu.sync_copy(acc.at[pl.ds(s_id * rpt, rpt)],
                        out_hbm.at[pl.ds(lo + s_id * rpt, rpt)])
        plsc.subcore_barrier()
        return 0

    my_chunks = (nchunks + 1 - c_id) // NC
    lax.fori_loop(0, my_chunks, _chunk, 0)


def _spmm(g, src, dst, wp, C, nchunks):
    f = pl.kernel(
        functools.partial(_spmm_body, wp, C, nchunks),
        out_type=jax.ShapeDtypeStruct((nchunks * C, wp), jnp.float32),
        mesh=_sc_mesh(),
        scratch_types=[
            pltpu.VMEM_SHARED((C + LL, wp), jnp.float32),
            pltpu.VMEM((EB,), jnp.int32),
            pltpu.VMEM((EB,), jnp.int32),
            pltpu.VMEM((G + LL,), jnp.int32),
            pltpu.VMEM((G + LL,), jnp.int32),
            pltpu.VMEM((2, G), jnp.int32),
            pltpu.VMEM((2, G), jnp.int32),
            pltpu.VMEM((2, G, wp), jnp.float32),
            pltpu.VMEM((16, wp), jnp.float32),
            pltpu.SemaphoreType.DMA,
            pltpu.SemaphoreType.DMA,
        ],
        compiler_params=pltpu.CompilerParams(needs_layout_passes=False, use_tc_tiling_on_sc=False),
    )
    return f(g, src, dst)


_R = 2000


def _reduce_body(h_ref, o_ref):
    deg = jnp.sum(h_ref[...], axis=0, keepdims=True) + 1.0
    o_ref[...] = lax.rsqrt(deg)


def _reduce_dinv(hists):
    out = pl.pallas_call(
        _reduce_body,
        out_shape=jax.ShapeDtypeStruct((1, NN), jnp.float32),
    )(hists)
    return jnp.reshape(out, (NN, 1))


def _l1_body(x_ref, d_ref, w_ref, o_ref):
    o_ref[...] = d_ref[...] * jnp.dot(
        x_ref[...], w_ref[...], preferred_element_type=jnp.float32)


def _layer1(xp, dinv, Wp, wp_in, wp_out):
    return pl.pallas_call(
        _l1_body,
        out_shape=jax.ShapeDtypeStruct((NN, wp_out), jnp.float32),
        grid=(NN // _R,),
        in_specs=[
            pl.BlockSpec((_R, wp_in), lambda i: (i, 0)),
            pl.BlockSpec((_R, 1), lambda i: (i, 0)),
            pl.BlockSpec((wp_in, wp_out), lambda i: (0, 0)),
        ],
        out_specs=pl.BlockSpec((_R, wp_out), lambda i: (i, 0)),
    )(xp, dinv, Wp)


def _mid_body(s_ref, g_ref, d_ref, b_ref, w_ref, o_ref):
    d = d_ref[...]
    pre = jnp.maximum(d * (s_ref[...] + g_ref[...]) + b_ref[...], 0.0)
    o_ref[...] = d * jnp.dot(pre, w_ref[...],
                             preferred_element_type=jnp.float32)


def _layer_mid(s, g, dinv, bp, Wp, wp_in, wp_out):
    return pl.pallas_call(
        _mid_body,
        out_shape=jax.ShapeDtypeStruct((NN, wp_out), jnp.float32),
        grid=(NN // _R,),
        in_specs=[
            pl.BlockSpec((_R, wp_in), lambda i: (i, 0)),
            pl.BlockSpec((_R, wp_in), lambda i: (i, 0)),
            pl.BlockSpec((_R, 1), lambda i: (i, 0)),
            pl.BlockSpec((1, wp_in), lambda i: (0, 0)),
            pl.BlockSpec((wp_in, wp_out), lambda i: (0, 0)),
        ],
        out_specs=pl.BlockSpec((_R, wp_out), lambda i: (i, 0)),
    )(s, g, dinv, bp, Wp)


def _final_body(w, s_ref, g_ref, d_ref, b_ref, z_ref, p_ref):
    z = jnp.maximum(d_ref[...] * (s_ref[...] + g_ref[...]) + b_ref[...], 0.0)
    wp = z.shape[1]
    msk = lax.broadcasted_iota(jnp.int32, (_R, wp), 1) < w
    zm = jnp.max(jnp.where(msk, z, -jnp.inf), axis=1, keepdims=True)
    ssum = jnp.sum(jnp.where(msk, jnp.exp(z - zm), 0.0), axis=1,
                   keepdims=True)
    z_ref[...] = z
    p_ref[...] = z - zm - jnp.log(ssum)


def _final(s, g, dinv, bp, w, wp_in):
    return pl.pallas_call(
        functools.partial(_final_body, w),
        out_shape=(jax.ShapeDtypeStruct((NN, wp_in), jnp.float32),
                   jax.ShapeDtypeStruct((NN, wp_in), jnp.float32)),
        grid=(NN // _R,),
        in_specs=[
            pl.BlockSpec((_R, wp_in), lambda i: (i, 0)),
            pl.BlockSpec((_R, wp_in), lambda i: (i, 0)),
            pl.BlockSpec((_R, 1), lambda i: (i, 0)),
            pl.BlockSpec((1, wp_in), lambda i: (0, 0)),
        ],
        out_specs=(pl.BlockSpec((_R, wp_in), lambda i: (i, 0)),
                   pl.BlockSpec((_R, wp_in), lambda i: (i, 0))),
    )(s, g, dinv, bp)


def _pad2(a, r, c):
    out = jnp.zeros((r, c), a.dtype)
    return out.at[:a.shape[0], :a.shape[1]].set(a)


_SPMM_GEOM = {176: (7168, 14), 96: (16768, 6), 48: (33408, 3),
              32: (50048, 2)}


def kernel(x, edge_index, train_flag, W1, b1, W3, b3, W4, b4, W2, b2):
    src = edge_index[0]
    dst = edge_index[1]

    hists = _hist(dst)
    dinv = _reduce_dinv(hists)

    c = 21
    dims = [21, 8 * c, 4 * c, 2 * c, c]
    wps = [32, 176, 96, 48, 32]
    Ws = [W1, W3, W4, W2]
    bs = [b1, b3, b4, b2]

    xp = _pad2(x, NN, wps[0])
    Wp = [_pad2(Ws[i], wps[i], wps[i + 1]) for i in range(4)]
    bp = [_pad2(bs[i][None, :], 1, wps[i + 1]) for i in range(4)]

    g = _layer1(xp, dinv, Wp[0], wps[0], wps[1])
    for li in (1, 2, 3):
        wp_in = wps[li]
        C, nch = _SPMM_GEOM[wp_in]
        s = _spmm(g, src, dst, wp_in, C, nch)[:NN]
        g = _layer_mid(s, g, dinv, bp[li - 1], Wp[li], wp_in, wps[li + 1])

    wp_in = wps[4]
    C, nch = _SPMM_GEOM[wp_in]
    s = _spmm(g, src, dst, wp_in, C, nch)[:NN]
    zfull, pfull = _final(s, g, dinv, bp[3], dims[4], wp_in)

    z = zfull[:, :dims[4]]
    p_z = pfull[:, :dims[4]]
    return (p_z, z, edge_index)


# speedup vs baseline: 14.1583x; 1.1242x over previous
import functools

import jax
import jax.numpy as jnp
from jax import lax
from jax.experimental import pallas as pl
from jax.experimental.pallas import tpu as pltpu
from jax.experimental.pallas import tpu_sc as plsc

NN = 100000
EE = 3200000
NC = 2
NS = 16
LL = 16

EB = 1600
HEB = 4000
G = 128
GRP = 10


def _sc_mesh():
    return plsc.VectorSubcoreMesh(core_axis_name="c", subcore_axis_name="s")


def _hist_body(dst_hbm, out_hbm, hist, dbuf):
    c_id = lax.axis_index("c")
    s_id = lax.axis_index("s")
    wid = s_id * NC + c_id
    zero = jnp.zeros((LL,), jnp.float32)
    one = jnp.ones((LL,), jnp.float32)

    def _z(i, _):
        hist[pl.ds(i * LL, LL)] = zero
        return 0
    lax.fori_loop(0, NN // LL, _z, 0)

    per_tile = EE // (NC * NS)

    def _blk(b, _):
        base = wid * per_tile + b * HEB
        pltpu.sync_copy(dst_hbm.at[pl.ds(base, HEB)], dbuf)

        def _vec(j, _):
            idx = dbuf[pl.ds(j * LL, LL)]
            plsc.addupdate_scatter(hist, [idx], one)
            return 0
        lax.fori_loop(0, HEB // LL, _vec, 0)
        return 0
    lax.fori_loop(0, per_tile // HEB, _blk, 0)

    pltpu.sync_copy(hist, out_hbm.at[wid])


def _hist(dst):
    f = pl.kernel(
        _hist_body,
        out_type=jax.ShapeDtypeStruct((NC * NS, NN), jnp.float32),
        mesh=_sc_mesh(),
        scratch_types=[
            pltpu.VMEM((NN,), jnp.float32),
            pltpu.VMEM((HEB,), jnp.int32),
        ],
        compiler_params=pltpu.CompilerParams(needs_layout_passes=False, use_tc_tiling_on_sc=False),
    )
    return f(dst)


def _spmm_body(wp, C, nchunks, g_hbm, src_hbm, dst_hbm, out_hbm,
               acc, dbuf, sbuf, csrc, cdst, fsrc, fdst, rows, zbuf,
               gsem, ssem):
    c_id = lax.axis_index("c")
    s_id = lax.axis_index("s")
    rpt = C // NS
    per_tile = EE // NS
    izero = jnp.zeros((LL,), jnp.int32)
    lanes = lax.broadcasted_iota(jnp.int32, (LL,), 0)

    def _zz(i, _):
        r = i // (wp // LL)
        k = i % (wp // LL)
        zbuf.at[r][pl.ds(k * LL, LL)] = jnp.zeros((LL,), jnp.float32)
        return 0
    lax.fori_loop(0, (16 * wp) // LL, _zz, 0)

    def _wait_gather():
        pltpu.make_async_copy(g_hbm.at[fsrc.at[0]], rows.at[0], gsem).wait()

    def _wait_scatter():
        pltpu.make_async_copy(rows.at[0], acc.at[fdst.at[0]], ssem).wait()

    def _start_batch(fc):
        p = fc & 1
        @pl.when(fc >= 2)
        def _():
            _wait_scatter()
        @pl.when(fc >= 1)
        def _():
            _wait_gather()
            q = 1 - p
            pltpu.make_async_copy(rows.at[q], acc.at[fdst.at[q]],
                                  ssem).start(add=True)
        for k in range(G // LL):
            fsrc.at[p][pl.ds(k * LL, LL)] = csrc[pl.ds(k * LL, LL)]
            fdst.at[p][pl.ds(k * LL, LL)] = cdst[pl.ds(k * LL, LL)]
        pltpu.make_async_copy(g_hbm.at[fsrc.at[p]], rows.at[p],
                              gsem).start()

    def _chunk(i, _):
        chunk = i * NC + c_id
        lo = chunk * C

        nz = (rpt + 15) // 16
        for zi in range(nz):
            cnt = min(16, rpt - zi * 16)
            pltpu.sync_copy(zbuf.at[pl.ds(0, cnt)],
                            acc.at[pl.ds(s_id * rpt + zi * 16, cnt)])
        @pl.when(s_id == 0)
        def _():
            pltpu.sync_copy(zbuf.at[pl.ds(0, LL)], acc.at[pl.ds(C, LL)])
        plsc.subcore_barrier()

        def _blk(b, carry):
            base = s_id * per_tile + b * EB
            pltpu.sync_copy(dst_hbm.at[pl.ds(base, EB)], dbuf)
            pltpu.sync_copy(src_hbm.at[pl.ds(base, EB)], sbuf)

            def _grp(t, carry):
                p, fc = carry
                for u in range(GRP):
                    j = t * GRP + u
                    d = dbuf[pl.ds(j * LL, LL)]
                    s = sbuf[pl.ds(j * LL, LL)]
                    m = (d >= lo) & (d < lo + C)
                    plsc.store_compressed(csrc.at[pl.ds(p, LL)], s, mask=m)
                    plsc.store_compressed(cdst.at[pl.ds(p, LL)], d - lo,
                                          mask=m)
                    p = p + plsc.all_reduce_population_count(m)[0]

                def _do(q, fc):
                    _start_batch(fc)
                    for k in range(GRP + 1):
                        csrc[pl.ds(k * LL, LL)] = csrc[pl.ds(G + k * LL, LL)]
                        cdst[pl.ds(k * LL, LL)] = cdst[pl.ds(G + k * LL, LL)]
                    return q - G, fc + 1
                return lax.cond(p >= G, _do, lambda q, fc: (q, fc), p, fc)
            return lax.fori_loop(0, EB // LL // GRP, _grp, carry)
        ptr, fc = lax.fori_loop(0, per_tile // EB, _blk, (0, 0))

        def _pad(jj, _):
            keep = (lanes + jj * LL) < ptr
            sv = csrc[pl.ds(jj * LL, LL)]
            dv = cdst[pl.ds(jj * LL, LL)]
            csrc[pl.ds(jj * LL, LL)] = jnp.where(keep, sv, izero)
            cdst[pl.ds(jj * LL, LL)] = jnp.where(keep, dv, izero + C)
            return 0
        lax.fori_loop(0, G // LL, _pad, 0)
        _start_batch(fc)
        @pl.when(fc >= 1)
        def _():
            _wait_scatter()
        _wait_gather()
        pf = fc & 1
        pltpu.make_async_copy(rows.at[pf], acc.at[fdst.at[pf]],
                              ssem).start(add=True)
        _wait_scatter()

        plsc.subcore_barrier()
        pltpu.sync_copy(acc.at[pl.ds(s_id * rpt, rpt)],
                        out_hbm.at[pl.ds(lo + s_id * rpt, rpt)])
        plsc.subcore_barrier()
        return 0

    my_chunks = (nchunks + 1 - c_id) // NC
    lax.fori_loop(0, my_chunks, _chunk, 0)


def _spmm(g, src, dst, wp, C, nchunks):
    f = pl.kernel(
        functools.partial(_spmm_body, wp, C, nchunks),
        out_type=jax.ShapeDtypeStruct((nchunks * C, wp), jnp.float32),
        mesh=_sc_mesh(),
        scratch_types=[
            pltpu.VMEM_SHARED((C + LL, wp), jnp.float32),
            pltpu.VMEM((EB,), jnp.int32),
            pltpu.VMEM((EB,), jnp.int32),
            pltpu.VMEM((G + (GRP + 1) * LL,), jnp.int32),
            pltpu.VMEM((G + (GRP + 1) * LL,), jnp.int32),
            pltpu.VMEM((2, G), jnp.int32),
            pltpu.VMEM((2, G), jnp.int32),
            pltpu.VMEM((2, G, wp), jnp.float32),
            pltpu.VMEM((16, wp), jnp.float32),
            pltpu.SemaphoreType.DMA,
            pltpu.SemaphoreType.DMA,
        ],
        compiler_params=pltpu.CompilerParams(needs_layout_passes=False, use_tc_tiling_on_sc=False),
    )
    return f(g, src, dst)


_R = 2000


def _reduce_body(h_ref, o_ref):
    deg = jnp.sum(h_ref[...], axis=0, keepdims=True) + 1.0
    o_ref[...] = lax.rsqrt(deg)


def _reduce_dinv(hists):
    out = pl.pallas_call(
        _reduce_body,
        out_shape=jax.ShapeDtypeStruct((1, NN), jnp.float32),
    )(hists)
    return jnp.reshape(out, (NN, 1))


def _l1_body(x_ref, d_ref, w_ref, o_ref):
    o_ref[...] = d_ref[...] * jnp.dot(
        x_ref[...], w_ref[...], preferred_element_type=jnp.float32)


def _layer1(xp, dinv, Wp, wp_in, wp_out):
    return pl.pallas_call(
        _l1_body,
        out_shape=jax.ShapeDtypeStruct((NN, wp_out), jnp.float32),
        grid=(NN // _R,),
        in_specs=[
            pl.BlockSpec((_R, wp_in), lambda i: (i, 0)),
            pl.BlockSpec((_R, 1), lambda i: (i, 0)),
            pl.BlockSpec((wp_in, wp_out), lambda i: (0, 0)),
        ],
        out_specs=pl.BlockSpec((_R, wp_out), lambda i: (i, 0)),
    )(xp, dinv, Wp)


def _mid_body(s_ref, g_ref, d_ref, b_ref, w_ref, o_ref):
    d = d_ref[...]
    pre = jnp.maximum(d * (s_ref[...] + g_ref[...]) + b_ref[...], 0.0)
    o_ref[...] = d * jnp.dot(pre, w_ref[...],
                             preferred_element_type=jnp.float32)


def _layer_mid(s, g, dinv, bp, Wp, wp_in, wp_out):
    return pl.pallas_call(
        _mid_body,
        out_shape=jax.ShapeDtypeStruct((NN, wp_out), jnp.float32),
        grid=(NN // _R,),
        in_specs=[
            pl.BlockSpec((_R, wp_in), lambda i: (i, 0)),
            pl.BlockSpec((_R, wp_in), lambda i: (i, 0)),
            pl.BlockSpec((_R, 1), lambda i: (i, 0)),
            pl.BlockSpec((1, wp_in), lambda i: (0, 0)),
            pl.BlockSpec((wp_in, wp_out), lambda i: (0, 0)),
        ],
        out_specs=pl.BlockSpec((_R, wp_out), lambda i: (i, 0)),
    )(s, g, dinv, bp, Wp)


def _final_body(w, s_ref, g_ref, d_ref, b_ref, z_ref, p_ref):
    z = jnp.maximum(d_ref[...] * (s_ref[...] + g_ref[...]) + b_ref[...], 0.0)
    wp = z.shape[1]
    msk = lax.broadcasted_iota(jnp.int32, (_R, wp), 1) < w
    zm = jnp.max(jnp.where(msk, z, -jnp.inf), axis=1, keepdims=True)
    ssum = jnp.sum(jnp.where(msk, jnp.exp(z - zm), 0.0), axis=1,
                   keepdims=True)
    z_ref[...] = z
    p_ref[...] = z - zm - jnp.log(ssum)


def _final(s, g, dinv, bp, w, wp_in):
    return pl.pallas_call(
        functools.partial(_final_body, w),
        out_shape=(jax.ShapeDtypeStruct((NN, wp_in), jnp.float32),
                   jax.ShapeDtypeStruct((NN, wp_in), jnp.float32)),
        grid=(NN // _R,),
        in_specs=[
            pl.BlockSpec((_R, wp_in), lambda i: (i, 0)),
            pl.BlockSpec((_R, wp_in), lambda i: (i, 0)),
            pl.BlockSpec((_R, 1), lambda i: (i, 0)),
            pl.BlockSpec((1, wp_in), lambda i: (0, 0)),
        ],
        out_specs=(pl.BlockSpec((_R, wp_in), lambda i: (i, 0)),
                   pl.BlockSpec((_R, wp_in), lambda i: (i, 0))),
    )(s, g, dinv, bp)


def _pad2(a, r, c):
    out = jnp.zeros((r, c), a.dtype)
    return out.at[:a.shape[0], :a.shape[1]].set(a)


_SPMM_GEOM = {176: (7040, 15), 96: (14336, 7), 48: (33408, 3),
              32: (50048, 2)}


def kernel(x, edge_index, train_flag, W1, b1, W3, b3, W4, b4, W2, b2):
    src = edge_index[0]
    dst = edge_index[1]

    hists = _hist(dst)
    dinv = _reduce_dinv(hists)

    c = 21
    dims = [21, 8 * c, 4 * c, 2 * c, c]
    wps = [32, 176, 96, 48, 32]
    Ws = [W1, W3, W4, W2]
    bs = [b1, b3, b4, b2]

    xp = _pad2(x, NN, wps[0])
    Wp = [_pad2(Ws[i], wps[i], wps[i + 1]) for i in range(4)]
    bp = [_pad2(bs[i][None, :], 1, wps[i + 1]) for i in range(4)]

    g = _layer1(xp, dinv, Wp[0], wps[0], wps[1])
    for li in (1, 2, 3):
        wp_in = wps[li]
        C, nch = _SPMM_GEOM[wp_in]
        s = _spmm(g, src, dst, wp_in, C, nch)[:NN]
        g = _layer_mid(s, g, dinv, bp[li - 1], Wp[li], wp_in, wps[li + 1])

    wp_in = wps[4]
    C, nch = _SPMM_GEOM[wp_in]
    s = _spmm(g, src, dst, wp_in, C, nch)[:NN]
    zfull, pfull = _final(s, g, dinv, bp[3], dims[4], wp_in)

    z = zfull[:, :dims[4]]
    p_z = pfull[:, :dims[4]]
    return (p_z, z, edge_index)


# speedup vs baseline: 20.6352x; 1.4575x over previous
import functools

import jax
import jax.numpy as jnp
from jax import lax
from jax.experimental import pallas as pl
from jax.experimental.pallas import tpu as pltpu
from jax.experimental.pallas import tpu_sc as plsc

NN = 100000
EE = 3200000
NC = 2
NS = 16
LL = 16

EB = 1600
HEB = 4000
G = 128
GRP = 10


def _sc_mesh():
    return plsc.VectorSubcoreMesh(core_axis_name="c", subcore_axis_name="s")


def _hist_body(dst_hbm, out_hbm, hist, dbuf):
    c_id = lax.axis_index("c")
    s_id = lax.axis_index("s")
    wid = s_id * NC + c_id
    zero = jnp.zeros((LL,), jnp.float32)
    one = jnp.ones((LL,), jnp.float32)

    def _z(i, _):
        hist[pl.ds(i * LL, LL)] = zero
        return 0
    lax.fori_loop(0, NN // LL, _z, 0)

    per_tile = EE // (NC * NS)

    def _blk(b, _):
        base = wid * per_tile + b * HEB
        pltpu.sync_copy(dst_hbm.at[pl.ds(base, HEB)], dbuf)

        def _vec(j, _):
            idx = dbuf[pl.ds(j * LL, LL)]
            plsc.addupdate_scatter(hist, [idx], one)
            return 0
        lax.fori_loop(0, HEB // LL, _vec, 0)
        return 0
    lax.fori_loop(0, per_tile // HEB, _blk, 0)

    pltpu.sync_copy(hist, out_hbm.at[wid])


def _hist(dst):
    f = pl.kernel(
        _hist_body,
        out_type=jax.ShapeDtypeStruct((NC * NS, NN), jnp.float32),
        mesh=_sc_mesh(),
        scratch_types=[
            pltpu.VMEM((NN,), jnp.float32),
            pltpu.VMEM((HEB,), jnp.int32),
        ],
        compiler_params=pltpu.CompilerParams(needs_layout_passes=False, use_tc_tiling_on_sc=False),
    )
    return f(dst)


def _spmm_body(wp, C, nchunks, g_hbm, src_hbm, dst_hbm, out_hbm,
               acc, dbuf, sbuf, csrc, cdst, fsrc, fdst, rows, zbuf,
               gsem, ssem):
    c_id = lax.axis_index("c")
    s_id = lax.axis_index("s")
    rpt = C // NS
    per_tile = EE // NS
    izero = jnp.zeros((LL,), jnp.int32)
    lanes = lax.broadcasted_iota(jnp.int32, (LL,), 0)

    def _zz(i, _):
        r = i // (wp // LL)
        k = i % (wp // LL)
        zbuf.at[r][pl.ds(k * LL, LL)] = jnp.zeros((LL,), jnp.float32)
        return 0
    lax.fori_loop(0, (16 * wp) // LL, _zz, 0)

    def _wait_gather():
        pltpu.make_async_copy(g_hbm.at[fsrc.at[0]], rows.at[0], gsem).wait()

    def _wait_scatter():
        pltpu.make_async_copy(rows.at[0], acc.at[fdst.at[0]], ssem).wait()

    def _start_batch(fc):
        p = fc & 1
        @pl.when(fc >= 2)
        def _():
            _wait_scatter()
        @pl.when(fc >= 1)
        def _():
            _wait_gather()
            q = 1 - p
            pltpu.make_async_copy(rows.at[q], acc.at[fdst.at[q]],
                                  ssem).start(add=True)
        for k in range(G // LL):
            fsrc.at[p][pl.ds(k * LL, LL)] = csrc[pl.ds(k * LL, LL)]
            fdst.at[p][pl.ds(k * LL, LL)] = cdst[pl.ds(k * LL, LL)]
        pltpu.make_async_copy(g_hbm.at[fsrc.at[p]], rows.at[p],
                              gsem).start()

    def _chunk(i, _):
        chunk = i * NC + c_id
        lo = chunk * C

        nz = (rpt + 15) // 16
        for zi in range(nz):
            cnt = min(16, rpt - zi * 16)
            pltpu.sync_copy(zbuf.at[pl.ds(0, cnt)],
                            acc.at[pl.ds(s_id * rpt + zi * 16, cnt)])
        @pl.when(s_id == 0)
        def _():
            pltpu.sync_copy(zbuf.at[pl.ds(0, LL)], acc.at[pl.ds(C, LL)])
        plsc.subcore_barrier()

        def _blk(b, carry):
            base = s_id * per_tile + b * EB
            pltpu.sync_copy(dst_hbm.at[pl.ds(base, EB)], dbuf)
            pltpu.sync_copy(src_hbm.at[pl.ds(base, EB)], sbuf)

            def _grp(t, carry):
                p, fc = carry
                for u in range(GRP):
                    j = t * GRP + u
                    d = dbuf[pl.ds(j * LL, LL)]
                    s = sbuf[pl.ds(j * LL, LL)]
                    m = (d >= lo) & (d < lo + C)
                    plsc.store_compressed(csrc.at[pl.ds(p, LL)], s, mask=m)
                    plsc.store_compressed(cdst.at[pl.ds(p, LL)], d - lo,
                                          mask=m)
                    p = p + plsc.all_reduce_population_count(m)[0]

                def _do(q, fc):
                    _start_batch(fc)
                    for k in range(GRP + 1):
                        csrc[pl.ds(k * LL, LL)] = csrc[pl.ds(G + k * LL, LL)]
                        cdst[pl.ds(k * LL, LL)] = cdst[pl.ds(G + k * LL, LL)]
                    return q - G, fc + 1
                return lax.cond(p >= G, _do, lambda q, fc: (q, fc), p, fc)
            return lax.fori_loop(0, EB // LL // GRP, _grp, carry)
        ptr, fc = lax.fori_loop(0, per_tile // EB, _blk, (0, 0))

        def _pad(jj, _):
            keep = (lanes + jj * LL) < ptr
            sv = csrc[pl.ds(jj * LL, LL)]
            dv = cdst[pl.ds(jj * LL, LL)]
            csrc[pl.ds(jj * LL, LL)] = jnp.where(keep, sv, izero)
            cdst[pl.ds(jj * LL, LL)] = jnp.where(keep, dv, izero + C)
            return 0
        lax.fori_loop(0, G // LL, _pad, 0)
        _start_batch(fc)
        @pl.when(fc >= 1)
        def _():
            _wait_scatter()
        _wait_gather()
        pf = fc & 1
        pltpu.make_async_copy(rows.at[pf], acc.at[fdst.at[pf]],
                              ssem).start(add=True)
        _wait_scatter()

        plsc.subcore_barrier()
        pltpu.sync_copy(acc.at[pl.ds(s_id * rpt, rpt)],
                        out_hbm.at[pl.ds(lo + s_id * rpt, rpt)])
        plsc.subcore_barrier()
        return 0

    my_chunks = (nchunks + 1 - c_id) // NC
    lax.fori_loop(0, my_chunks, _chunk, 0)


def _spmm(g, src, dst, wp, C, nchunks):
    f = pl.kernel(
        functools.partial(_spmm_body, wp, C, nchunks),
        out_type=jax.ShapeDtypeStruct((nchunks * C, wp), jnp.float32),
        mesh=_sc_mesh(),
        scratch_types=[
            pltpu.VMEM_SHARED((C + LL, wp), jnp.float32),
            pltpu.VMEM((EB,), jnp.int32),
            pltpu.VMEM((EB,), jnp.int32),
            pltpu.VMEM((G + (GRP + 1) * LL,), jnp.int32),
            pltpu.VMEM((G + (GRP + 1) * LL,), jnp.int32),
            pltpu.VMEM((2, G), jnp.int32),
            pltpu.VMEM((2, G), jnp.int32),
            pltpu.VMEM((2, G, wp), jnp.float32),
            pltpu.VMEM((16, wp), jnp.float32),
            pltpu.SemaphoreType.DMA,
            pltpu.SemaphoreType.DMA,
        ],
        compiler_params=pltpu.CompilerParams(needs_layout_passes=False, use_tc_tiling_on_sc=False),
    )
    return f(g, src, dst)


_R = 2000


def _reduce_body(h_ref, o_ref):
    deg = jnp.sum(h_ref[...], axis=0, keepdims=True) + 1.0
    o_ref[...] = lax.rsqrt(deg)


def _reduce_dinv(hists):
    out = pl.pallas_call(
        _reduce_body,
        out_shape=jax.ShapeDtypeStruct((1, NN), jnp.float32),
    )(hists)
    return jnp.reshape(out, (NN, 1))


def _scale_body(x_ref, d_ref, o_ref):
    o_ref[...] = d_ref[...] * x_ref[...]


def _scale(xp, dinv, wp):
    return pl.pallas_call(
        _scale_body,
        out_shape=jax.ShapeDtypeStruct((NN, wp), jnp.float32),
        grid=(NN // _R,),
        in_specs=[
            pl.BlockSpec((_R, wp), lambda i: (i, 0)),
            pl.BlockSpec((_R, 1), lambda i: (i, 0)),
        ],
        out_specs=pl.BlockSpec((_R, wp), lambda i: (i, 0)),
    )(xp, dinv)


def _first_body(s_ref, u_ref, d_ref, b_ref, w1_ref, w3_ref, o_ref):
    d = d_ref[...]
    h = jnp.dot(s_ref[...] + u_ref[...], w1_ref[...],
                preferred_element_type=jnp.float32)
    pre = jnp.maximum(d * h + b_ref[...], 0.0)
    o_ref[...] = d * jnp.dot(pre, w3_ref[...],
                             preferred_element_type=jnp.float32)


def _layer_first(s, u, dinv, bp, W1p, W3p, wp_in, wp_mid, wp_out):
    return pl.pallas_call(
        _first_body,
        out_shape=jax.ShapeDtypeStruct((NN, wp_out), jnp.float32),
        grid=(NN // _R,),
        in_specs=[
            pl.BlockSpec((_R, wp_in), lambda i: (i, 0)),
            pl.BlockSpec((_R, wp_in), lambda i: (i, 0)),
            pl.BlockSpec((_R, 1), lambda i: (i, 0)),
            pl.BlockSpec((1, wp_mid), lambda i: (0, 0)),
            pl.BlockSpec((wp_in, wp_mid), lambda i: (0, 0)),
            pl.BlockSpec((wp_mid, wp_out), lambda i: (0, 0)),
        ],
        out_specs=pl.BlockSpec((_R, wp_out), lambda i: (i, 0)),
    )(s, u, dinv, bp, W1p, W3p)


def _l1_body(x_ref, d_ref, w_ref, o_ref):
    o_ref[...] = d_ref[...] * jnp.dot(
        x_ref[...], w_ref[...], preferred_element_type=jnp.float32)


def _layer1(xp, dinv, Wp, wp_in, wp_out):
    return pl.pallas_call(
        _l1_body,
        out_shape=jax.ShapeDtypeStruct((NN, wp_out), jnp.float32),
        grid=(NN // _R,),
        in_specs=[
            pl.BlockSpec((_R, wp_in), lambda i: (i, 0)),
            pl.BlockSpec((_R, 1), lambda i: (i, 0)),
            pl.BlockSpec((wp_in, wp_out), lambda i: (0, 0)),
        ],
        out_specs=pl.BlockSpec((_R, wp_out), lambda i: (i, 0)),
    )(xp, dinv, Wp)


def _mid_body(s_ref, g_ref, d_ref, b_ref, w_ref, o_ref):
    d = d_ref[...]
    pre = jnp.maximum(d * (s_ref[...] + g_ref[...]) + b_ref[...], 0.0)
    o_ref[...] = d * jnp.dot(pre, w_ref[...],
                             preferred_element_type=jnp.float32)


def _layer_mid(s, g, dinv, bp, Wp, wp_in, wp_out):
    return pl.pallas_call(
        _mid_body,
        out_shape=jax.ShapeDtypeStruct((NN, wp_out), jnp.float32),
        grid=(NN // _R,),
        in_specs=[
            pl.BlockSpec((_R, wp_in), lambda i: (i, 0)),
            pl.BlockSpec((_R, wp_in), lambda i: (i, 0)),
            pl.BlockSpec((_R, 1), lambda i: (i, 0)),
            pl.BlockSpec((1, wp_in), lambda i: (0, 0)),
            pl.BlockSpec((wp_in, wp_out), lambda i: (0, 0)),
        ],
        out_specs=pl.BlockSpec((_R, wp_out), lambda i: (i, 0)),
    )(s, g, dinv, bp, Wp)


def _final_body(w, s_ref, g_ref, d_ref, b_ref, z_ref, p_ref):
    z = jnp.maximum(d_ref[...] * (s_ref[...] + g_ref[...]) + b_ref[...], 0.0)
    wp = z.shape[1]
    msk = lax.broadcasted_iota(jnp.int32, (_R, wp), 1) < w
    zm = jnp.max(jnp.where(msk, z, -jnp.inf), axis=1, keepdims=True)
    ssum = jnp.sum(jnp.where(msk, jnp.exp(z - zm), 0.0), axis=1,
                   keepdims=True)
    z_ref[...] = z
    p_ref[...] = z - zm - jnp.log(ssum)


def _final(s, g, dinv, bp, w, wp_in):
    return pl.pallas_call(
        functools.partial(_final_body, w),
        out_shape=(jax.ShapeDtypeStruct((NN, wp_in), jnp.float32),
                   jax.ShapeDtypeStruct((NN, wp_in), jnp.float32)),
        grid=(NN // _R,),
        in_specs=[
            pl.BlockSpec((_R, wp_in), lambda i: (i, 0)),
            pl.BlockSpec((_R, wp_in), lambda i: (i, 0)),
            pl.BlockSpec((_R, 1), lambda i: (i, 0)),
            pl.BlockSpec((1, wp_in), lambda i: (0, 0)),
        ],
        out_specs=(pl.BlockSpec((_R, wp_in), lambda i: (i, 0)),
                   pl.BlockSpec((_R, wp_in), lambda i: (i, 0))),
    )(s, g, dinv, bp)


def _pad2(a, r, c):
    out = jnp.zeros((r, c), a.dtype)
    return out.at[:a.shape[0], :a.shape[1]].set(a)


_SPMM_GEOM = {176: (7040, 15), 96: (14336, 7), 48: (33408, 3),
              32: (50048, 2)}


def kernel(x, edge_index, train_flag, W1, b1, W3, b3, W4, b4, W2, b2):
    src = edge_index[0]
    dst = edge_index[1]

    hists = _hist(dst)
    dinv = _reduce_dinv(hists)

    c = 21
    dims = [21, 8 * c, 4 * c, 2 * c, c]
    wps = [32, 176, 96, 48, 32]
    Ws = [W1, W3, W4, W2]
    bs = [b1, b3, b4, b2]

    xp = _pad2(x, NN, wps[0])
    Wp = [_pad2(Ws[i], wps[i], wps[i + 1]) for i in range(4)]
    bp = [_pad2(bs[i][None, :], 1, wps[i + 1]) for i in range(4)]

    u = _scale(xp, dinv, wps[0])
    C, nch = _SPMM_GEOM[wps[0]]
    s = _spmm(u, src, dst, wps[0], C, nch)[:NN]
    g = _layer_first(s, u, dinv, bp[0], Wp[0], Wp[1], wps[0], wps[1],
                     wps[2])
    for li in (2, 3):
        wp_in = wps[li]
        C, nch = _SPMM_GEOM[wp_in]
        s = _spmm(g, src, dst, wp_in, C, nch)[:NN]
        g = _layer_mid(s, g, dinv, bp[li - 1], Wp[li], wp_in, wps[li + 1])

    wp_in = wps[4]
    C, nch = _SPMM_GEOM[wp_in]
    s = _spmm(g, src, dst, wp_in, C, nch)[:NN]
    zfull, pfull = _final(s, g, dinv, bp[3], dims[4], wp_in)

    z = zfull[:, :dims[4]]
    p_z = pfull[:, :dims[4]]
    return (p_z, z, edge_index)


# speedup vs baseline: 22.7722x; 1.1036x over previous
import functools

import jax
import jax.numpy as jnp
from jax import lax
from jax.experimental import pallas as pl
from jax.experimental.pallas import tpu as pltpu
from jax.experimental.pallas import tpu_sc as plsc

NN = 100000
EE = 3200000
NC = 2
NS = 16
LL = 16

EB = 1600
HEB = 4000
G = 128
GRP = 10


def _sc_mesh():
    return plsc.VectorSubcoreMesh(core_axis_name="c", subcore_axis_name="s")


def _hist_body(dst_hbm, out_hbm, hist, dbuf):
    c_id = lax.axis_index("c")
    s_id = lax.axis_index("s")
    wid = s_id * NC + c_id
    zero = jnp.zeros((LL,), jnp.float32)
    one = jnp.ones((LL,), jnp.float32)

    def _z(i, _):
        hist[pl.ds(i * LL, LL)] = zero
        return 0
    lax.fori_loop(0, NN // LL, _z, 0)

    per_tile = EE // (NC * NS)

    def _blk(b, _):
        base = wid * per_tile + b * HEB
        pltpu.sync_copy(dst_hbm.at[pl.ds(base, HEB)], dbuf)

        def _vec(j, _):
            idx = dbuf[pl.ds(j * LL, LL)]
            plsc.addupdate_scatter(hist, [idx], one)
            return 0
        lax.fori_loop(0, HEB // LL, _vec, 0)
        return 0
    lax.fori_loop(0, per_tile // HEB, _blk, 0)

    pltpu.sync_copy(hist, out_hbm.at[wid])


def _hist(dst):
    f = pl.kernel(
        _hist_body,
        out_type=jax.ShapeDtypeStruct((NC * NS, NN), jnp.float32),
        mesh=_sc_mesh(),
        scratch_types=[
            pltpu.VMEM((NN,), jnp.float32),
            pltpu.VMEM((HEB,), jnp.int32),
        ],
        compiler_params=pltpu.CompilerParams(needs_layout_passes=False, use_tc_tiling_on_sc=False),
    )
    return f(dst)


def _spmm_body(wp, C, nchunks, g_hbm, src_hbm, dst_hbm, out_hbm,
               acc, dbuf, sbuf, csrc, cdst, fsrc, fdst, rows, zbuf,
               gsem, ssem):
    c_id = lax.axis_index("c")
    s_id = lax.axis_index("s")
    rpt = C // NS
    per_tile = EE // NS
    izero = jnp.zeros((LL,), jnp.int32)
    lanes = lax.broadcasted_iota(jnp.int32, (LL,), 0)

    def _zz(i, _):
        r = i // (wp // LL)
        k = i % (wp // LL)
        zbuf.at[r][pl.ds(k * LL, LL)] = jnp.zeros((LL,), jnp.float32)
        return 0
    lax.fori_loop(0, (16 * wp) // LL, _zz, 0)

    def _wait_gather():
        pltpu.make_async_copy(g_hbm.at[fsrc.at[0]], rows.at[0], gsem).wait()

    def _wait_scatter():
        pltpu.make_async_copy(rows.at[0], acc.at[fdst.at[0]], ssem).wait()

    def _start_batch(fc):
        p = fc & 1
        @pl.when(fc >= 2)
        def _():
            _wait_scatter()
        @pl.when(fc >= 1)
        def _():
            _wait_gather()
            q = 1 - p
            pltpu.make_async_copy(rows.at[q], acc.at[fdst.at[q]],
                                  ssem).start(add=True)
        for k in range(G // LL):
            fsrc.at[p][pl.ds(k * LL, LL)] = csrc[pl.ds(k * LL, LL)]
            fdst.at[p][pl.ds(k * LL, LL)] = cdst[pl.ds(k * LL, LL)]
        pltpu.make_async_copy(g_hbm.at[fsrc.at[p]], rows.at[p],
                              gsem).start()

    def _chunk(i, _):
        chunk = i * NC + c_id
        lo = chunk * C

        nz = (rpt + 15) // 16
        for zi in range(nz):
            cnt = min(16, rpt - zi * 16)
            pltpu.sync_copy(zbuf.at[pl.ds(0, cnt)],
                            acc.at[pl.ds(s_id * rpt + zi * 16, cnt)])
        @pl.when(s_id == 0)
        def _():
            pltpu.sync_copy(zbuf.at[pl.ds(0, LL)], acc.at[pl.ds(C, LL)])
        plsc.subcore_barrier()

        def _blk(b, carry):
            base = s_id * per_tile + b * EB
            pltpu.sync_copy(dst_hbm.at[pl.ds(base, EB)], dbuf)
            pltpu.sync_copy(src_hbm.at[pl.ds(base, EB)], sbuf)

            def _grp(t, carry):
                p, fc = carry
                for u in range(GRP):
                    j = t * GRP + u
                    d = dbuf[pl.ds(j * LL, LL)]
                    s = sbuf[pl.ds(j * LL, LL)]
                    m = (d >= lo) & (d < lo + C)
                    plsc.store_compressed(csrc.at[pl.ds(p, LL)], s, mask=m)
                    plsc.store_compressed(cdst.at[pl.ds(p, LL)], d - lo,
                                          mask=m)
                    p = p + plsc.all_reduce_population_count(m)[0]

                def _do(q, fc):
                    _start_batch(fc)
                    for k in range(GRP + 1):
                        csrc[pl.ds(k * LL, LL)] = csrc[pl.ds(G + k * LL, LL)]
                        cdst[pl.ds(k * LL, LL)] = cdst[pl.ds(G + k * LL, LL)]
                    return q - G, fc + 1
                return lax.cond(p >= G, _do, lambda q, fc: (q, fc), p, fc)
            return lax.fori_loop(0, EB // LL // GRP, _grp, carry)
        ptr, fc = lax.fori_loop(0, per_tile // EB, _blk, (0, 0))

        def _pad(jj, _):
            keep = (lanes + jj * LL) < ptr
            sv = csrc[pl.ds(jj * LL, LL)]
            dv = cdst[pl.ds(jj * LL, LL)]
            csrc[pl.ds(jj * LL, LL)] = jnp.where(keep, sv, izero)
            cdst[pl.ds(jj * LL, LL)] = jnp.where(keep, dv, izero + C)
            return 0
        lax.fori_loop(0, G // LL, _pad, 0)
        _start_batch(fc)
        @pl.when(fc >= 1)
        def _():
            _wait_scatter()
        _wait_gather()
        pf = fc & 1
        pltpu.make_async_copy(rows.at[pf], acc.at[fdst.at[pf]],
                              ssem).start(add=True)
        _wait_scatter()

        plsc.subcore_barrier()
        pltpu.sync_copy(acc.at[pl.ds(s_id * rpt, rpt)],
                        out_hbm.at[pl.ds(lo + s_id * rpt, rpt)])
        plsc.subcore_barrier()
        return 0

    my_chunks = (nchunks + 1 - c_id) // NC
    lax.fori_loop(0, my_chunks, _chunk, 0)


def _spmm(g, src, dst, wp, C, nchunks):
    f = pl.kernel(
        functools.partial(_spmm_body, wp, C, nchunks),
        out_type=jax.ShapeDtypeStruct((nchunks * C, wp), jnp.float32),
        mesh=_sc_mesh(),
        scratch_types=[
            pltpu.VMEM_SHARED((C + LL, wp), jnp.float32),
            pltpu.VMEM((EB,), jnp.int32),
            pltpu.VMEM((EB,), jnp.int32),
            pltpu.VMEM((G + (GRP + 1) * LL,), jnp.int32),
            pltpu.VMEM((G + (GRP + 1) * LL,), jnp.int32),
            pltpu.VMEM((2, G), jnp.int32),
            pltpu.VMEM((2, G), jnp.int32),
            pltpu.VMEM((2, G, wp), jnp.float32),
            pltpu.VMEM((16, wp), jnp.float32),
            pltpu.SemaphoreType.DMA,
            pltpu.SemaphoreType.DMA,
        ],
        compiler_params=pltpu.CompilerParams(needs_layout_passes=False, use_tc_tiling_on_sc=False),
    )
    return f(g, src, dst)


_R = 2000


def _reduce_body(h_ref, o_ref):
    deg = jnp.sum(h_ref[...], axis=0, keepdims=True) + 1.0
    o_ref[...] = lax.rsqrt(deg)


def _reduce_dinv(hists):
    out = pl.pallas_call(
        _reduce_body,
        out_shape=jax.ShapeDtypeStruct((1, NN), jnp.float32),
    )(hists)
    return jnp.reshape(out, (NN, 1))


def _scale_body(x_ref, d_ref, o_ref):
    o_ref[...] = d_ref[...] * x_ref[...]


def _scale(xp, dinv, wp):
    return pl.pallas_call(
        _scale_body,
        out_shape=jax.ShapeDtypeStruct((NN, wp), jnp.float32),
        grid=(NN // _R,),
        in_specs=[
            pl.BlockSpec((_R, wp), lambda i: (i, 0)),
            pl.BlockSpec((_R, 1), lambda i: (i, 0)),
        ],
        out_specs=pl.BlockSpec((_R, wp), lambda i: (i, 0)),
    )(xp, dinv)


def _first_body(s_ref, u_ref, d_ref, b_ref, w1_ref, w3_ref, o_ref):
    d = d_ref[...]
    h = jnp.dot(s_ref[...] + u_ref[...], w1_ref[...],
                preferred_element_type=jnp.float32)
    pre = jnp.maximum(d * h + b_ref[...], 0.0)
    o_ref[...] = d * jnp.dot(pre, w3_ref[...],
                             preferred_element_type=jnp.float32)


def _layer_first(s, u, dinv, bp, W1p, W3p, wp_in, wp_mid, wp_out):
    return pl.pallas_call(
        _first_body,
        out_shape=jax.ShapeDtypeStruct((NN, wp_out), jnp.float32),
        grid=(NN // _R,),
        in_specs=[
            pl.BlockSpec((_R, wp_in), lambda i: (i, 0)),
            pl.BlockSpec((_R, wp_in), lambda i: (i, 0)),
            pl.BlockSpec((_R, 1), lambda i: (i, 0)),
            pl.BlockSpec((1, wp_mid), lambda i: (0, 0)),
            pl.BlockSpec((wp_in, wp_mid), lambda i: (0, 0)),
            pl.BlockSpec((wp_mid, wp_out), lambda i: (0, 0)),
        ],
        out_specs=pl.BlockSpec((_R, wp_out), lambda i: (i, 0)),
    )(s, u, dinv, bp, W1p, W3p)


def _l1_body(x_ref, d_ref, w_ref, o_ref):
    o_ref[...] = d_ref[...] * jnp.dot(
        x_ref[...], w_ref[...], preferred_element_type=jnp.float32)


def _layer1(xp, dinv, Wp, wp_in, wp_out):
    return pl.pallas_call(
        _l1_body,
        out_shape=jax.ShapeDtypeStruct((NN, wp_out), jnp.float32),
        grid=(NN // _R,),
        in_specs=[
            pl.BlockSpec((_R, wp_in), lambda i: (i, 0)),
            pl.BlockSpec((_R, 1), lambda i: (i, 0)),
            pl.BlockSpec((wp_in, wp_out), lambda i: (0, 0)),
        ],
        out_specs=pl.BlockSpec((_R, wp_out), lambda i: (i, 0)),
    )(xp, dinv, Wp)


def _mid_body(s_ref, g_ref, d_ref, b_ref, w_ref, o_ref):
    d = d_ref[...]
    pre = jnp.maximum(d * (s_ref[...] + g_ref[...]) + b_ref[...], 0.0)
    o_ref[...] = d * jnp.dot(pre, w_ref[...],
                             preferred_element_type=jnp.float32)


def _layer_mid(s, g, dinv, bp, Wp, wp_in, wp_out):
    return pl.pallas_call(
        _mid_body,
        out_shape=jax.ShapeDtypeStruct((NN, wp_out), jnp.float32),
        grid=(NN // _R,),
        in_specs=[
            pl.BlockSpec((_R, wp_in), lambda i: (i, 0)),
            pl.BlockSpec((_R, wp_in), lambda i: (i, 0)),
            pl.BlockSpec((_R, 1), lambda i: (i, 0)),
            pl.BlockSpec((1, wp_in), lambda i: (0, 0)),
            pl.BlockSpec((wp_in, wp_out), lambda i: (0, 0)),
        ],
        out_specs=pl.BlockSpec((_R, wp_out), lambda i: (i, 0)),
    )(s, g, dinv, bp, Wp)


def _final_body(w, s_ref, g_ref, d_ref, b_ref, z_ref, p_ref):
    z = jnp.maximum(d_ref[...] * (s_ref[...] + g_ref[...]) + b_ref[...], 0.0)
    wp = z.shape[1]
    msk = lax.broadcasted_iota(jnp.int32, (_R, wp), 1) < w
    zm = jnp.max(jnp.where(msk, z, -jnp.inf), axis=1, keepdims=True)
    ssum = jnp.sum(jnp.where(msk, jnp.exp(z - zm), 0.0), axis=1,
                   keepdims=True)
    z_ref[...] = z
    p_ref[...] = z - zm - jnp.log(ssum)


def _final(s, g, dinv, bp, w, wp_in):
    return pl.pallas_call(
        functools.partial(_final_body, w),
        out_shape=(jax.ShapeDtypeStruct((NN, wp_in), jnp.float32),
                   jax.ShapeDtypeStruct((NN, wp_in), jnp.float32)),
        grid=(NN // _R,),
        in_specs=[
            pl.BlockSpec((_R, wp_in), lambda i: (i, 0)),
            pl.BlockSpec((_R, wp_in), lambda i: (i, 0)),
            pl.BlockSpec((_R, 1), lambda i: (i, 0)),
            pl.BlockSpec((1, wp_in), lambda i: (0, 0)),
        ],
        out_specs=(pl.BlockSpec((_R, wp_in), lambda i: (i, 0)),
                   pl.BlockSpec((_R, wp_in), lambda i: (i, 0))),
    )(s, g, dinv, bp)


def _pad2(a, r, c):
    out = jnp.zeros((r, c), a.dtype)
    return out.at[:a.shape[0], :a.shape[1]].set(a)


_SPMM_GEOM = {176: (7040, 15), 96: (16752, 6), 48: (25024, 4),
              32: (50048, 2)}


def kernel(x, edge_index, train_flag, W1, b1, W3, b3, W4, b4, W2, b2):
    src = edge_index[0]
    dst = edge_index[1]

    hists = _hist(dst)
    dinv = _reduce_dinv(hists)

    c = 21
    dims = [21, 8 * c, 4 * c, 2 * c, c]
    wps = [32, 176, 96, 48, 32]
    Ws = [W1, W3, W4, W2]
    bs = [b1, b3, b4, b2]

    xp = _pad2(x, NN, wps[0])
    Wp = [_pad2(Ws[i], wps[i], wps[i + 1]) for i in range(4)]
    bp = [_pad2(bs[i][None, :], 1, wps[i + 1]) for i in range(4)]

    u = _scale(xp, dinv, wps[0])
    C, nch = _SPMM_GEOM[wps[0]]
    s = _spmm(u, src, dst, wps[0], C, nch)[:NN]
    g = _layer_first(s, u, dinv, bp[0], Wp[0], Wp[1], wps[0], wps[1],
                     wps[2])
    for li in (2, 3):
        wp_in = wps[li]
        C, nch = _SPMM_GEOM[wp_in]
        s = _spmm(g, src, dst, wp_in, C, nch)[:NN]
        g = _layer_mid(s, g, dinv, bp[li - 1], Wp[li], wp_in, wps[li + 1])

    wp_in = wps[4]
    C, nch = _SPMM_GEOM[wp_in]
    s = _spmm(g, src, dst, wp_in, C, nch)[:NN]
    zfull, pfull = _final(s, g, dinv, bp[3], dims[4], wp_in)

    z = zfull[:, :dims[4]]
    p_z = pfull[:, :dims[4]]
    return (p_z, z, edge_index)


# speedup vs baseline: 25.7694x; 1.1316x over previous
import functools

import jax
import jax.numpy as jnp
from jax import lax
from jax.experimental import pallas as pl
from jax.experimental.pallas import tpu as pltpu
from jax.experimental.pallas import tpu_sc as plsc

NN = 100000
EE = 3200000
NC = 2
NS = 16
LL = 16

EB = 1600
HEB = 4000
GRP = 10


def _sc_mesh():
    return plsc.VectorSubcoreMesh(core_axis_name="c", subcore_axis_name="s")


def _hist_body(dst_hbm, out_hbm, hist, dbuf):
    c_id = lax.axis_index("c")
    s_id = lax.axis_index("s")
    wid = s_id * NC + c_id
    zero = jnp.zeros((LL,), jnp.float32)
    one = jnp.ones((LL,), jnp.float32)

    def _z(i, _):
        hist[pl.ds(i * LL, LL)] = zero
        return 0
    lax.fori_loop(0, NN // LL, _z, 0)

    per_tile = EE // (NC * NS)

    def _blk(b, _):
        base = wid * per_tile + b * HEB
        pltpu.sync_copy(dst_hbm.at[pl.ds(base, HEB)], dbuf)

        def _vec(j, _):
            idx = dbuf[pl.ds(j * LL, LL)]
            plsc.addupdate_scatter(hist, [idx], one)
            return 0
        lax.fori_loop(0, HEB // LL, _vec, 0)
        return 0
    lax.fori_loop(0, per_tile // HEB, _blk, 0)

    pltpu.sync_copy(hist, out_hbm.at[wid])


def _hist(dst):
    f = pl.kernel(
        _hist_body,
        out_type=jax.ShapeDtypeStruct((NC * NS, NN), jnp.float32),
        mesh=_sc_mesh(),
        scratch_types=[
            pltpu.VMEM((NN,), jnp.float32),
            pltpu.VMEM((HEB,), jnp.int32),
        ],
        compiler_params=pltpu.CompilerParams(needs_layout_passes=False, use_tc_tiling_on_sc=False),
    )
    return f(dst)


def _spmm_body(wp, C, nchunks, G, g_hbm, src_hbm, dst_hbm, out_hbm,
               acc, dbuf, sbuf, csrc, cdst, fsrc, fdst, rows, zbuf,
               gsem, ssem):
    c_id = lax.axis_index("c")
    s_id = lax.axis_index("s")
    rpt = C // NS
    per_tile = EE // NS
    izero = jnp.zeros((LL,), jnp.int32)
    lanes = lax.broadcasted_iota(jnp.int32, (LL,), 0)

    def _zz(i, _):
        r = i // (wp // LL)
        k = i % (wp // LL)
        zbuf.at[r][pl.ds(k * LL, LL)] = jnp.zeros((LL,), jnp.float32)
        return 0
    lax.fori_loop(0, (16 * wp) // LL, _zz, 0)

    def _wait_gather():
        pltpu.make_async_copy(g_hbm.at[fsrc.at[0]], rows.at[0], gsem).wait()

    def _wait_scatter():
        pltpu.make_async_copy(rows.at[0], acc.at[fdst.at[0]], ssem).wait()

    def _start_batch(fc):
        p = fc & 1
        @pl.when(fc >= 2)
        def _():
            _wait_scatter()
        @pl.when(fc >= 1)
        def _():
            _wait_gather()
            q = 1 - p
            pltpu.make_async_copy(rows.at[q], acc.at[fdst.at[q]],
                                  ssem).start(add=True)
        for k in range(G // LL):
            fsrc.at[p][pl.ds(k * LL, LL)] = csrc[pl.ds(k * LL, LL)]
            fdst.at[p][pl.ds(k * LL, LL)] = cdst[pl.ds(k * LL, LL)]
        pltpu.make_async_copy(g_hbm.at[fsrc.at[p]], rows.at[p],
                              gsem).start()

    def _chunk(i, _):
        chunk = i * NC + c_id
        lo = chunk * C

        nz = (rpt + 15) // 16
        for zi in range(nz):
            cnt = min(16, rpt - zi * 16)
            pltpu.sync_copy(zbuf.at[pl.ds(0, cnt)],
                            acc.at[pl.ds(s_id * rpt + zi * 16, cnt)])
        @pl.when(s_id == 0)
        def _():
            pltpu.sync_copy(zbuf.at[pl.ds(0, LL)], acc.at[pl.ds(C, LL)])
        plsc.subcore_barrier()

        def _blk(b, carry):
            base = s_id * per_tile + b * EB
            pltpu.sync_copy(dst_hbm.at[pl.ds(base, EB)], dbuf)
            pltpu.sync_copy(src_hbm.at[pl.ds(base, EB)], sbuf)

            def _grp(t, carry):
                p, fc = carry
                for u in range(GRP):
                    j = t * GRP + u
                    d = dbuf[pl.ds(j * LL, LL)]
                    s = sbuf[pl.ds(j * LL, LL)]
                    m = (d >= lo) & (d < lo + C)
                    plsc.store_compressed(csrc.at[pl.ds(p, LL)], s, mask=m)
                    plsc.store_compressed(cdst.at[pl.ds(p, LL)], d - lo,
                                          mask=m)
                    p = p + plsc.all_reduce_population_count(m)[0]

                def _do(q, fc):
                    _start_batch(fc)
                    for k in range(GRP + 1):
                        csrc[pl.ds(k * LL, LL)] = csrc[pl.ds(G + k * LL, LL)]
                        cdst[pl.ds(k * LL, LL)] = cdst[pl.ds(G + k * LL, LL)]
                    return q - G, fc + 1
                return lax.cond(p >= G, _do, lambda q, fc: (q, fc), p, fc)
            return lax.fori_loop(0, EB // LL // GRP, _grp, carry)
        ptr, fc = lax.fori_loop(0, per_tile // EB, _blk, (0, 0))

        def _pad(jj, _):
            keep = (lanes + jj * LL) < ptr
            sv = csrc[pl.ds(jj * LL, LL)]
            dv = cdst[pl.ds(jj * LL, LL)]
            csrc[pl.ds(jj * LL, LL)] = jnp.where(keep, sv, izero)
            cdst[pl.ds(jj * LL, LL)] = jnp.where(keep, dv, izero + C)
            return 0
        lax.fori_loop(0, G // LL, _pad, 0)
        _start_batch(fc)
        @pl.when(fc >= 1)
        def _():
            _wait_scatter()
        _wait_gather()
        pf = fc & 1
        pltpu.make_async_copy(rows.at[pf], acc.at[fdst.at[pf]],
                              ssem).start(add=True)
        _wait_scatter()

        plsc.subcore_barrier()
        pltpu.sync_copy(acc.at[pl.ds(s_id * rpt, rpt)],
                        out_hbm.at[pl.ds(lo + s_id * rpt, rpt)])
        plsc.subcore_barrier()
        return 0

    my_chunks = (nchunks + 1 - c_id) // NC
    lax.fori_loop(0, my_chunks, _chunk, 0)


def _spmm(g, src, dst, wp, C, nchunks, G):
    f = pl.kernel(
        functools.partial(_spmm_body, wp, C, nchunks, G),
        out_type=jax.ShapeDtypeStruct((nchunks * C, wp), jnp.float32),
        mesh=_sc_mesh(),
        scratch_types=[
            pltpu.VMEM_SHARED((C + LL, wp), jnp.float32),
            pltpu.VMEM((EB,), jnp.int32),
            pltpu.VMEM((EB,), jnp.int32),
            pltpu.VMEM((G + (GRP + 1) * LL,), jnp.int32),
            pltpu.VMEM((G + (GRP + 1) * LL,), jnp.int32),
            pltpu.VMEM((2, G), jnp.int32),
            pltpu.VMEM((2, G), jnp.int32),
            pltpu.VMEM((2, G, wp), jnp.float32),
            pltpu.VMEM((16, wp), jnp.float32),
            pltpu.SemaphoreType.DMA,
            pltpu.SemaphoreType.DMA,
        ],
        compiler_params=pltpu.CompilerParams(needs_layout_passes=False, use_tc_tiling_on_sc=False),
    )
    return f(g, src, dst)


_R = 2000


def _reduce_body(h_ref, o_ref):
    deg = jnp.sum(h_ref[...], axis=0, keepdims=True) + 1.0
    o_ref[...] = lax.rsqrt(deg)


def _reduce_dinv(hists):
    out = pl.pallas_call(
        _reduce_body,
        out_shape=jax.ShapeDtypeStruct((1, NN), jnp.float32),
    )(hists)
    return jnp.reshape(out, (NN, 1))


def _scale_body(x_ref, d_ref, o_ref):
    o_ref[...] = d_ref[...] * x_ref[...]


def _scale(xp, dinv, wp):
    return pl.pallas_call(
        _scale_body,
        out_shape=jax.ShapeDtypeStruct((NN, wp), jnp.float32),
        grid=(NN // _R,),
        in_specs=[
            pl.BlockSpec((_R, wp), lambda i: (i, 0)),
            pl.BlockSpec((_R, 1), lambda i: (i, 0)),
        ],
        out_specs=pl.BlockSpec((_R, wp), lambda i: (i, 0)),
    )(xp, dinv)


def _first_body(s_ref, u_ref, d_ref, b_ref, w1_ref, w3_ref, o_ref):
    d = d_ref[...]
    h = jnp.dot(s_ref[...] + u_ref[...], w1_ref[...],
                preferred_element_type=jnp.float32)
    pre = jnp.maximum(d * h + b_ref[...], 0.0)
    o_ref[...] = d * jnp.dot(pre, w3_ref[...],
                             preferred_element_type=jnp.float32)


def _layer_first(s, u, dinv, bp, W1p, W3p, wp_in, wp_mid, wp_out):
    return pl.pallas_call(
        _first_body,
        out_shape=jax.ShapeDtypeStruct((NN, wp_out), jnp.float32),
        grid=(NN // _R,),
        in_specs=[
            pl.BlockSpec((_R, wp_in), lambda i: (i, 0)),
            pl.BlockSpec((_R, wp_in), lambda i: (i, 0)),
            pl.BlockSpec((_R, 1), lambda i: (i, 0)),
            pl.BlockSpec((1, wp_mid), lambda i: (0, 0)),
            pl.BlockSpec((wp_in, wp_mid), lambda i: (0, 0)),
            pl.BlockSpec((wp_mid, wp_out), lambda i: (0, 0)),
        ],
        out_specs=pl.BlockSpec((_R, wp_out), lambda i: (i, 0)),
    )(s, u, dinv, bp, W1p, W3p)


def _l1_body(x_ref, d_ref, w_ref, o_ref):
    o_ref[...] = d_ref[...] * jnp.dot(
        x_ref[...], w_ref[...], preferred_element_type=jnp.float32)


def _layer1(xp, dinv, Wp, wp_in, wp_out):
    return pl.pallas_call(
        _l1_body,
        out_shape=jax.ShapeDtypeStruct((NN, wp_out), jnp.float32),
        grid=(NN // _R,),
        in_specs=[
            pl.BlockSpec((_R, wp_in), lambda i: (i, 0)),
            pl.BlockSpec((_R, 1), lambda i: (i, 0)),
            pl.BlockSpec((wp_in, wp_out), lambda i: (0, 0)),
        ],
        out_specs=pl.BlockSpec((_R, wp_out), lambda i: (i, 0)),
    )(xp, dinv, Wp)


def _mid_body(s_ref, g_ref, d_ref, b_ref, w_ref, o_ref):
    d = d_ref[...]
    pre = jnp.maximum(d * (s_ref[...] + g_ref[...]) + b_ref[...], 0.0)
    o_ref[...] = d * jnp.dot(pre, w_ref[...],
                             preferred_element_type=jnp.float32)


def _layer_mid(s, g, dinv, bp, Wp, wp_in, wp_out):
    return pl.pallas_call(
        _mid_body,
        out_shape=jax.ShapeDtypeStruct((NN, wp_out), jnp.float32),
        grid=(NN // _R,),
        in_specs=[
            pl.BlockSpec((_R, wp_in), lambda i: (i, 0)),
            pl.BlockSpec((_R, wp_in), lambda i: (i, 0)),
            pl.BlockSpec((_R, 1), lambda i: (i, 0)),
            pl.BlockSpec((1, wp_in), lambda i: (0, 0)),
            pl.BlockSpec((wp_in, wp_out), lambda i: (0, 0)),
        ],
        out_specs=pl.BlockSpec((_R, wp_out), lambda i: (i, 0)),
    )(s, g, dinv, bp, Wp)


def _final_body(w, s_ref, g_ref, d_ref, b_ref, z_ref, p_ref):
    z = jnp.maximum(d_ref[...] * (s_ref[...] + g_ref[...]) + b_ref[...], 0.0)
    wp = z.shape[1]
    msk = lax.broadcasted_iota(jnp.int32, (_R, wp), 1) < w
    zm = jnp.max(jnp.where(msk, z, -jnp.inf), axis=1, keepdims=True)
    ssum = jnp.sum(jnp.where(msk, jnp.exp(z - zm), 0.0), axis=1,
                   keepdims=True)
    z_ref[...] = z
    p_ref[...] = z - zm - jnp.log(ssum)


def _final(s, g, dinv, bp, w, wp_in):
    return pl.pallas_call(
        functools.partial(_final_body, w),
        out_shape=(jax.ShapeDtypeStruct((NN, wp_in), jnp.float32),
                   jax.ShapeDtypeStruct((NN, wp_in), jnp.float32)),
        grid=(NN // _R,),
        in_specs=[
            pl.BlockSpec((_R, wp_in), lambda i: (i, 0)),
            pl.BlockSpec((_R, wp_in), lambda i: (i, 0)),
            pl.BlockSpec((_R, 1), lambda i: (i, 0)),
            pl.BlockSpec((1, wp_in), lambda i: (0, 0)),
        ],
        out_specs=(pl.BlockSpec((_R, wp_in), lambda i: (i, 0)),
                   pl.BlockSpec((_R, wp_in), lambda i: (i, 0))),
    )(s, g, dinv, bp)


def _pad2(a, r, c):
    out = jnp.zeros((r, c), a.dtype)
    return out.at[:a.shape[0], :a.shape[1]].set(a)


_SPMM_GEOM = {176: (7040, 15, 128), 96: (16752, 6, 128),
              48: (25024, 4, 256), 32: (50048, 2, 256)}


def kernel(x, edge_index, train_flag, W1, b1, W3, b3, W4, b4, W2, b2):
    src = edge_index[0]
    dst = edge_index[1]

    hists = _hist(dst)
    dinv = _reduce_dinv(hists)

    c = 21
    dims = [21, 8 * c, 4 * c, 2 * c, c]
    wps = [32, 176, 96, 48, 32]
    Ws = [W1, W3, W4, W2]
    bs = [b1, b3, b4, b2]

    xp = _pad2(x, NN, wps[0])
    Wp = [_pad2(Ws[i], wps[i], wps[i + 1]) for i in range(4)]
    bp = [_pad2(bs[i][None, :], 1, wps[i + 1]) for i in range(4)]

    u = _scale(xp, dinv, wps[0])
    C, nch, Gv = _SPMM_GEOM[wps[0]]
    s = _spmm(u, src, dst, wps[0], C, nch, Gv)[:NN]
    g = _layer_first(s, u, dinv, bp[0], Wp[0], Wp[1], wps[0], wps[1],
                     wps[2])
    for li in (2, 3):
        wp_in = wps[li]
        C, nch, Gv = _SPMM_GEOM[wp_in]
        s = _spmm(g, src, dst, wp_in, C, nch, Gv)[:NN]
        g = _layer_mid(s, g, dinv, bp[li - 1], Wp[li], wp_in, wps[li + 1])

    wp_in = wps[4]
    C, nch, Gv = _SPMM_GEOM[wp_in]
    s = _spmm(g, src, dst, wp_in, C, nch, Gv)[:NN]
    zfull, pfull = _final(s, g, dinv, bp[3], dims[4], wp_in)

    z = zfull[:, :dims[4]]
    p_z = pfull[:, :dims[4]]
    return (p_z, z, edge_index)
